# baseline (speedup 1.0000x reference)
import functools

import jax
import jax.numpy as jnp
from jax import lax
from jax.experimental import pallas as pl
from jax.experimental.pallas import tpu as pltpu
from jax.experimental.pallas import tpu_sc as plsc

f32 = jnp.float32
i32 = jnp.int32

NU, NI = 25000, 25000
N = NU + NI
D = 64
E = 800000
NBATCH = 16384
LAYERS = 4
NC, NS, L = 2, 16, 16
HALF = 25000
HPAD = 25088
RPT = HPAD // NS
NPAD = 2 * HPAD
TRASH = HALF
CH = 128
NCHT = 392
EPT = NCHT * CH
EPAD = NS * EPT
GP = 56
NOUTP = NCHT // GP
GL = 8
SENT = 1 << 30
RB = 32
NRB = RPT // RB
BPT = NBATCH // (NC * NS)
VT = NC * NS
EPV = EPAD // VT
SB = 6272
FQ = 1024
NCH2 = 224
PADOFF = HPAD - HALF

_mesh = plsc.VectorSubcoreMesh(core_axis_name="c", subcore_axis_name="s",
                               num_cores=NC, num_subcores=NS)


def _rsqrt16(x):
    x = jnp.maximum(x, 1.0)
    ii = lax.bitcast_convert_type(x, i32)
    ii = 0x5F3759DF - lax.shift_right_logical(ii, 1)
    y = lax.bitcast_convert_type(ii, f32)
    for _ in range(3):
        y = y * (1.5 - 0.5 * x * y * y)
    return y


def _bcast(w, j):
    return lax.broadcast_in_dim(lax.slice_in_dim(w, j, j + 1), (L,), (0,))


@functools.partial(
    pl.kernel,
    out_type=(jax.ShapeDtypeStruct((NC, VT, NCH2, CH), i32),
              jax.ShapeDtypeStruct((NC, VT, NCH2, CH), i32),
              jax.ShapeDtypeStruct((NC, VT, L), i32)),
    mesh=_mesh,
    compiler_params=pltpu.CompilerParams(use_tc_tiling_on_sc=False,
                                         needs_layout_passes=False),
    scratch_types=(
        pltpu.VMEM((SB,), i32),
        pltpu.VMEM((SB,), i32),
        pltpu.VMEM((2064,), i32),
        pltpu.VMEM((2064,), i32),
        pltpu.VMEM((2064,), i32),
        pltpu.VMEM((2064,), i32),
        pltpu.VMEM((L,), i32),
        pltpu.SemaphoreType.DMA,
    ),
)
def _prep_k(row_hbm, col_hbm, lrow, lcadj, counts,
            rst, cst, olr, olc, ohr, ohc, cnt16, fsem):
    cid = lax.axis_index("c")
    sid = lax.axis_index("s")
    v = cid * NS + sid
    lane = lax.iota(i32, L)
    trash16 = TRASH + (lane & 7)
    QC = FQ // CH

    def flush(side, rbuf, cbuf, fb):
        for k in range(QC):
            pltpu.async_copy(rbuf.at[pl.ds(k * CH, CH)],
                             lrow.at[side, v, fb * QC + k], fsem)
            pltpu.async_copy(cbuf.at[pl.ds(k * CH, CH)],
                             lcadj.at[side, v, fb * QC + k], fsem)
        for k in range(QC):
            pltpu.make_async_copy(rbuf.at[pl.ds(k * CH, CH)],
                                  lrow.at[side, v, 0], fsem).wait()
            pltpu.make_async_copy(cbuf.at[pl.ds(k * CH, CH)],
                                  lcadj.at[side, v, 0], fsem).wait()
        rbuf[pl.ds(0, L)] = rbuf[pl.ds(FQ, L)]
        cbuf[pl.ds(0, L)] = cbuf[pl.ds(FQ, L)]

    def outer(og, carry):
        eb = v * EPV + og * SB
        pltpu.sync_copy(row_hbm.at[pl.ds(eb, SB)], rst)
        pltpu.sync_copy(col_hbm.at[pl.ds(eb, SB)], cst)

        def blk(i, car):
            wpl, fbl, wph, fbh = car
            r = rst[pl.ds(i * L, L)]
            c = cst[pl.ds(i * L, L)]
            mlo = r < HALF
            rhi = jnp.where(r >= (SENT // 2), trash16, r - HALF)
            cadjv = jnp.where(c >= (SENT // 2), 0,
                              jnp.where(c >= HALF, c + PADOFF, c))
            plsc.store_compressed(olr.at[pl.ds(wpl, L)], r, mask=mlo)
            plsc.store_compressed(olc.at[pl.ds(wpl, L)], cadjv, mask=mlo)
            mhi = jnp.logical_not(mlo)
            plsc.store_compressed(ohr.at[pl.ds(wph, L)], rhi, mask=mhi)
            plsc.store_compressed(ohc.at[pl.ds(wph, L)], cadjv, mask=mhi)
            nl = plsc.all_reduce_population_count(mlo)[0]
            wpl = wpl + nl
            wph = wph + (L - nl)
            filled_lo = wpl >= FQ
            filled_hi = wph >= FQ

            @pl.when(filled_lo)
            def _():
                flush(0, olr, olc, fbl)

            @pl.when(filled_hi)
            def _():
                flush(1, ohr, ohc, fbh)

            wpl = jnp.where(filled_lo, wpl - FQ, wpl)
            fbl = fbl + jnp.where(filled_lo, 1, 0)
            wph = jnp.where(filled_hi, wph - FQ, wph)
            fbh = fbh + jnp.where(filled_hi, 1, 0)
            return (wpl, fbl, wph, fbh)

        return lax.fori_loop(0, SB // L, blk, carry)

    wpl, fbl, wph, fbh = lax.fori_loop(
        0, EPV // SB, outer,
        (jnp.array(0, i32), jnp.array(0, i32),
         jnp.array(0, i32), jnp.array(0, i32)))

    def tail(side, rbuf, cbuf, wp, fb):
        def fill(i, _):
            rbuf[pl.ds(wp + i * L, L)] = trash16
            cbuf[pl.ds(wp + i * L, L)] = jnp.zeros((L,), i32)
            return 0

        lax.fori_loop(0, FQ // L, fill, 0)
        flush(side, rbuf, cbuf, fb)
        cnt16[...] = jnp.full((L,), (fb + 1) * QC, i32)
        pltpu.sync_copy(cnt16, counts.at[side, v])

    tail(0, olr, olc, wpl, fbl)
    tail(1, ohr, ohc, wph, fbh)


@functools.partial(
    pl.kernel,
    out_type=(jax.ShapeDtypeStruct((NPAD,), f32),
              jax.ShapeDtypeStruct((NPAD,), f32),
              jax.ShapeDtypeStruct((NPAD, D), f32)),
    mesh=_mesh,
    compiler_params=pltpu.CompilerParams(use_tc_tiling_on_sc=False),
    scratch_types=(
        pltpu.VMEM_SHARED((HPAD,), f32),
        pltpu.VMEM_SHARED((HPAD,), f32),
        pltpu.VMEM((GP * CH,), i32),
        pltpu.VMEM((GP * CH,), i32),
        pltpu.VMEM((GP, CH), i32),
        pltpu.VMEM((GP, CH), i32),
        pltpu.VMEM((CH,), f32),
        pltpu.VMEM((RPT,), f32),
        pltpu.VMEM((RPT,), f32),
        pltpu.VMEM((RPT,), f32),
        pltpu.VMEM((RPT,), f32),
        pltpu.VMEM((RPT,), f32),
        pltpu.VMEM((RB, D), f32),
        pltpu.VMEM((RB, D), f32),
    ),
)
def _deg_k(row_hbm, col_hbm, emb, dinv_r, dinv_c, g0,
           histr, histc, rst, cst, lr, lc, ones, zb, hrb, hcb, drb, dcb,
           eb, gb):
    cid = lax.axis_index("c")
    sid = lax.axis_index("s")
    base = cid * HALF
    rb0 = sid * RPT

    def z16(i, _):
        zb[pl.ds(i * L, L)] = jnp.zeros((L,), f32)
        return 0

    lax.fori_loop(0, RPT // L, z16, 0)
    pltpu.sync_copy(zb, histr.at[pl.ds(rb0, RPT)])
    pltpu.sync_copy(zb, histc.at[pl.ds(rb0, RPT)])
    for k in range(CH // L):
        ones[pl.ds(k * L, L)] = jnp.ones((L,), f32)
    plsc.subcore_barrier()

    def outer(o, _):
        eb_ = sid * EPT + o * (GP * CH)
        pltpu.sync_copy(row_hbm.at[pl.ds(eb_, GP * CH)], rst)
        pltpu.sync_copy(col_hbm.at[pl.ds(eb_, GP * CH)], cst)

        def inner(g2, _):
            for k in range(CH // L):
                r = rst[pl.ds(g2 * CH + k * L, L)]
                ro = (r >= base) & (r < base + HALF)
                lr[g2, pl.ds(k * L, L)] = jnp.where(
                    ro, r - base, TRASH + (r & 7))
                c = cst[pl.ds(g2 * CH + k * L, L)]
                co = (c >= base) & (c < base + HALF)
                lc[g2, pl.ds(k * L, L)] = jnp.where(
                    co, c - base, TRASH + (c & 7))
            return 0

        lax.fori_loop(0, GP, inner, 0)

        def scat(j, _):
            pltpu.sync_copy(ones, histr.at[lr.at[j]], add=True)
            pltpu.sync_copy(ones, histc.at[lc.at[j]], add=True)
            return 0

        lax.fori_loop(0, GP, scat, 0)
        return 0

    lax.fori_loop(0, NOUTP, outer, 0)
    plsc.subcore_barrier()

    pltpu.sync_copy(histr.at[pl.ds(rb0, RPT)], hrb)
    pltpu.sync_copy(histc.at[pl.ds(rb0, RPT)], hcb)

    def dblk(b, _):
        drb[pl.ds(b * L, L)] = _rsqrt16(hrb[pl.ds(b * L, L)])
        dcb[pl.ds(b * L, L)] = _rsqrt16(hcb[pl.ds(b * L, L)])
        return 0

    lax.fori_loop(0, RPT // L, dblk, 0)
    pltpu.sync_copy(drb, dinv_r.at[pl.ds(cid * HPAD + rb0, RPT)])
    pltpu.sync_copy(dcb, dinv_c.at[pl.ds(cid * HPAD + rb0, RPT)])

    def go(o, _):
        rowb = rb0 + o * RB
        grow = cid * HPAD + rowb
        pltpu.sync_copy(emb.at[pl.ds(grow, RB)], eb)

        def blkfn(b2, _):
            roff = b2 * L
            wc = dcb[pl.ds(o * RB + roff, L)]
            for j in range(L):
                dcv = _bcast(wc, j)
                ro = roff + j
                for q in range(D // L):
                    gb[ro, pl.ds(q * L, L)] = eb[ro, pl.ds(q * L, L)] * dcv
            return 0

        lax.fori_loop(0, RB // L, blkfn, 0)
        pltpu.sync_copy(gb, g0.at[pl.ds(grow, RB)])
        return 0

    lax.fori_loop(0, NRB, go, 0)


@functools.partial(
    pl.kernel,
    out_type=(jax.ShapeDtypeStruct((NPAD, D), f32),
              jax.ShapeDtypeStruct((NPAD, D), f32)),
    mesh=_mesh,
    compiler_params=pltpu.CompilerParams(use_tc_tiling_on_sc=False),
    scratch_types=(
        pltpu.VMEM_SHARED((HPAD, D), f32),
        pltpu.VMEM((2, GL, CH), i32),
        pltpu.VMEM((2, GL, CH), i32),
        pltpu.VMEM((2, CH, D), f32),
        pltpu.VMEM((RB, D), f32),
        pltpu.VMEM((RB, D), f32),
        pltpu.VMEM((RPT,), f32),
        pltpu.VMEM((RPT,), f32),
        pltpu.VMEM((L,), i32),
        pltpu.SemaphoreType.DMA,
        pltpu.SemaphoreType.DMA,
        pltpu.SemaphoreType.DMA,
    ),
)
def _layer_k(lrow, lcadj, counts, gin, sin, dr, dc, gout, sout,
             acc, lst, ast, gbuf, ab, sb, drb, dcb, cntb,
             gsem0, gsem1, isem):
    cid = lax.axis_index("c")
    sid = lax.axis_index("s")
    rb0 = sid * RPT

    def zrow(i, _):
        for q in range(D // L):
            ab[i, pl.ds(q * L, L)] = jnp.zeros((L,), f32)
        return 0

    lax.fori_loop(0, RB, zrow, 0)

    def zcp(o, _):
        pltpu.sync_copy(ab, acc.at[pl.ds(rb0 + o * RB, RB)])
        return 0

    lax.fori_loop(0, NRB, zcp, 0)
    plsc.subcore_barrier()

    for dv in range(2):
        vsl = 2 * sid + dv
        pltpu.sync_copy(counts.at[cid, vsl], cntb)
        ngrp = cntb[pl.ds(0, L)][0] // GL
        pltpu.async_copy(lrow.at[cid, vsl, pl.ds(0, GL)], lst.at[0], isem)
        pltpu.async_copy(lcadj.at[cid, vsl, pl.ds(0, GL)], ast.at[0], isem)

        def outer(o, _):
            slot = lax.rem(o, 2)
            pltpu.make_async_copy(lrow.at[cid, vsl, pl.ds(0, GL)],
                                  lst.at[slot], isem).wait()
            pltpu.make_async_copy(lcadj.at[cid, vsl, pl.ds(0, GL)],
                                  ast.at[slot], isem).wait()

            @pl.when(o + 1 < ngrp)
            def _():
                nb = (o + 1) * GL
                pltpu.async_copy(lrow.at[cid, vsl, pl.ds(nb, GL)],
                                 lst.at[1 - slot], isem)
                pltpu.async_copy(lcadj.at[cid, vsl, pl.ds(nb, GL)],
                                 ast.at[1 - slot], isem)

            pltpu.async_copy(gin.at[ast.at[slot, 0]], gbuf.at[0], gsem0)
            pltpu.async_copy(gin.at[ast.at[slot, 1]], gbuf.at[1], gsem1)

            def wait_scatter_fire(j, gs, sem, fire):
                pltpu.make_async_copy(gin.at[ast.at[slot, j]], gbuf.at[gs],
                                      sem).wait()
                pltpu.sync_copy(gbuf.at[gs], acc.at[lst.at[slot, j]],
                                add=True)
                if fire:
                    pltpu.async_copy(gin.at[ast.at[slot, j + 2]],
                                     gbuf.at[gs], sem)

            def ej(j2, _):
                wait_scatter_fire(2 * j2, 0, gsem0, True)
                wait_scatter_fire(2 * j2 + 1, 1, gsem1, True)
                return 0

            lax.fori_loop(0, (GL - 2) // 2, ej, 0)
            wait_scatter_fire(GL - 2, 0, gsem0, False)
            wait_scatter_fire(GL - 1, 1, gsem1, False)
            return 0

        lax.fori_loop(0, ngrp, outer, 0)
    plsc.subcore_barrier()

    pltpu.sync_copy(dr.at[pl.ds(cid * HPAD + rb0, RPT)], drb)
    pltpu.sync_copy(dc.at[pl.ds(cid * HPAD + rb0, RPT)], dcb)

    def no(o, _):
        rowb = rb0 + o * RB
        grow = cid * HPAD + rowb
        pltpu.sync_copy(acc.at[pl.ds(rowb, RB)], ab)
        pltpu.sync_copy(sin.at[pl.ds(grow, RB)], sb)

        def blkfn(b2, _):
            roff = b2 * L
            wr = drb[pl.ds(o * RB + roff, L)]
            wc = dcb[pl.ds(o * RB + roff, L)]
            for j in range(L):
                brv = _bcast(wr, j)
                brc = brv * _bcast(wc, j)
                ro = roff + j
                for q in range(D // L):
                    s = ab[ro, pl.ds(q * L, L)]
                    ab[ro, pl.ds(q * L, L)] = s * brc
                    sb[ro, pl.ds(q * L, L)] = sb[ro, pl.ds(q * L, L)] + s * brv
            return 0

        lax.fori_loop(0, RB // L, blkfn, 0)
        pltpu.sync_copy(ab, gout.at[pl.ds(grow, RB)])
        pltpu.sync_copy(sb, sout.at[pl.ds(grow, RB)])
        return 0

    lax.fori_loop(0, NRB, no, 0)


@functools.partial(
    pl.kernel,
    out_type=jax.ShapeDtypeStruct((NBATCH,), f32),
    mesh=_mesh,
    compiler_params=pltpu.CompilerParams(use_tc_tiling_on_sc=False),
    scratch_types=(
        pltpu.VMEM((BPT,), i32),
        pltpu.VMEM((BPT,), i32),
        pltpu.VMEM((BPT // CH, CH), i32),
        pltpu.VMEM((BPT // CH, CH), i32),
        pltpu.VMEM((CH, D), f32),
        pltpu.VMEM((CH, D), f32),
        pltpu.VMEM((BPT,), f32),
        pltpu.VMEM((2 * L,), f32),
    ),
)
def _gamma_k(users, items, sfin, gamma, ust, ist, ub2, ib2, ur, ir, ob, tb):
    cid = lax.axis_index("c")
    sid = lax.axis_index("s")
    wid = cid * NS + sid
    bb = wid * BPT
    pltpu.sync_copy(users.at[pl.ds(bb, BPT)], ust)
    pltpu.sync_copy(items.at[pl.ds(bb, BPT)], ist)

    def re(i, _):
        j2 = i // (CH // L)
        k = i % (CH // L)
        ub2[j2, pl.ds(k * L, L)] = ust[pl.ds(i * L, L)]
        ib2[j2, pl.ds(k * L, L)] = ist[pl.ds(i * L, L)] + HPAD
        return 0

    lax.fori_loop(0, BPT // L, re, 0)
    lane = lax.iota(i32, L)
    tb[pl.ds(L, L)] = jnp.zeros((L,), f32)

    def jf(j, _):
        pltpu.sync_copy(sfin.at[ub2.at[j]], ur)
        pltpu.sync_copy(sfin.at[ib2.at[j]], ir)

        def grp(g, _):
            out16 = jnp.zeros((L,), f32)
            for e in range(L):
                ro = g * L + e
                acc = ur[ro, pl.ds(0, L)] * ir[ro, pl.ds(0, L)]
                for q in range(1, D // L):
                    acc = acc + ur[ro, pl.ds(q * L, L)] * ir[ro, pl.ds(q * L, L)]
                for sh in (8, 4, 2, 1):
                    tb[pl.ds(0, L)] = acc
                    acc = acc + tb[pl.ds(sh, L)]
                dsum = _bcast(acc, 0) * (1.0 / 25.0)
                out16 = jnp.where(lane == e, dsum, out16)
            ob[pl.ds(j * CH + g * L, L)] = out16
            return 0

        lax.fori_loop(0, CH // L, grp, 0)
        return 0

    lax.fori_loop(0, BPT // CH, jf, 0)
    pltpu.sync_copy(ob, gamma.at[pl.ds(bb, BPT)])


def kernel(users, items, user_emb, item_emb, edge_index):
    row = edge_index[0].astype(i32)
    col = edge_index[1].astype(i32)
    pad = jnp.full((EPAD - E,), SENT, i32)
    row_p = jnp.concatenate([row, pad])
    col_p = jnp.concatenate([col, pad])
    zpad = jnp.zeros((HPAD - HALF, D), f32)
    emb = jnp.concatenate([user_emb, zpad, item_emb, zpad], axis=0)
    lrow, lcadj, counts = _prep_k(row_p, col_p)
    dinv_r, dinv_c, g0 = _deg_k(row_p, col_p, emb)
    g, s = g0, emb
    for _ in range(LAYERS):
        g, s = _layer_k(lrow, lcadj, counts, g, s, dinv_r, dinv_c)
    return _gamma_k(users.astype(i32), items.astype(i32), s)


# speedup vs baseline: 2.2552x; 2.2552x over previous
import functools

import jax
import jax.numpy as jnp
from jax import lax
from jax.experimental import pallas as pl
from jax.experimental.pallas import tpu as pltpu
from jax.experimental.pallas import tpu_sc as plsc

f32 = jnp.float32
i32 = jnp.int32

NU, NI = 25000, 25000
N = NU + NI
D = 64
E = 800000
NBATCH = 16384
LAYERS = 4
NC, NS, L = 2, 16, 16
HALF = 25000
HPAD = 25088
RPT = HPAD // NS
NPAD = 2 * HPAD
TRASH = HALF
CH = 128
NCHT = 392
EPT = NCHT * CH
EPAD = NS * EPT
GP = 56
NOUTP = NCHT // GP
GL = 8
NOUTL = NCHT // GL
SENT = 1 << 30
RB = 32
NRB = RPT // RB
BPT = NBATCH // (NC * NS)

_mesh = plsc.VectorSubcoreMesh(core_axis_name="c", subcore_axis_name="s",
                               num_cores=NC, num_subcores=NS)


def _rsqrt16(x):
    x = jnp.maximum(x, 1.0)
    ii = lax.bitcast_convert_type(x, i32)
    ii = 0x5F3759DF - lax.shift_right_logical(ii, 1)
    y = lax.bitcast_convert_type(ii, f32)
    for _ in range(3):
        y = y * (1.5 - 0.5 * x * y * y)
    return y


def _bcast(w, j):
    return lax.broadcast_in_dim(lax.slice_in_dim(w, j, j + 1), (L,), (0,))


@functools.partial(
    pl.kernel,
    out_type=(jax.ShapeDtypeStruct((NC, NS * NCHT, CH), i32),
              jax.ShapeDtypeStruct((NS * NCHT, CH), i32)),
    mesh=_mesh,
    compiler_params=pltpu.CompilerParams(use_tc_tiling_on_sc=False),
    scratch_types=(
        pltpu.VMEM((GP * CH,), i32),
        pltpu.VMEM((GP * CH,), i32),
        pltpu.VMEM((GP, CH), i32),
        pltpu.VMEM((GP, CH), i32),
    ),
)
def _prep_k(row_hbm, col_hbm, rloc, cadj, rst, cst, lst, ast):
    cid = lax.axis_index("c")
    sid = lax.axis_index("s")
    base = cid * HALF

    def outer(o, _):
        eb = sid * EPT + o * (GP * CH)
        pltpu.sync_copy(row_hbm.at[pl.ds(eb, GP * CH)], rst)
        pltpu.sync_copy(col_hbm.at[pl.ds(eb, GP * CH)], cst)

        def inner(g2, _):
            for k in range(CH // L):
                r = rst[pl.ds(g2 * CH + k * L, L)]
                owned = (r >= base) & (r < base + HALF)
                lst[g2, pl.ds(k * L, L)] = jnp.where(
                    owned, r - base, TRASH + (r & 7))
                c = cst[pl.ds(g2 * CH + k * L, L)]
                cc = jnp.where(c >= HALF, c + (HPAD - HALF), c)
                ast[g2, pl.ds(k * L, L)] = jnp.where(c >= SENT // 2, 0, cc)
            return 0

        lax.fori_loop(0, GP, inner, 0)
        rb = sid * NCHT + o * GP
        pltpu.sync_copy(lst, rloc.at[cid, pl.ds(rb, GP)])

        @pl.when(cid == 0)
        def _():
            pltpu.sync_copy(ast, cadj.at[pl.ds(rb, GP)])

        return 0

    lax.fori_loop(0, NOUTP, outer, 0)


@functools.partial(
    pl.kernel,
    out_type=(jax.ShapeDtypeStruct((NPAD,), f32),
              jax.ShapeDtypeStruct((NPAD,), f32),
              jax.ShapeDtypeStruct((NPAD, D), f32)),
    mesh=_mesh,
    compiler_params=pltpu.CompilerParams(use_tc_tiling_on_sc=False),
    scratch_types=(
        pltpu.VMEM_SHARED((HPAD,), f32),
        pltpu.VMEM_SHARED((HPAD,), f32),
        pltpu.VMEM((GP * CH,), i32),
        pltpu.VMEM((GP * CH,), i32),
        pltpu.VMEM((GP, CH), i32),
        pltpu.VMEM((GP, CH), i32),
        pltpu.VMEM((CH,), f32),
        pltpu.VMEM((RPT,), f32),
        pltpu.VMEM((RPT,), f32),
        pltpu.VMEM((RPT,), f32),
        pltpu.VMEM((RPT,), f32),
        pltpu.VMEM((RPT,), f32),
        pltpu.VMEM((RB, D), f32),
        pltpu.VMEM((RB, D), f32),
    ),
)
def _deg_k(row_hbm, col_hbm, emb, dinv_r, dinv_c, g0,
           histr, histc, rst, cst, lr, lc, ones, zb, hrb, hcb, drb, dcb,
           eb, gb):
    cid = lax.axis_index("c")
    sid = lax.axis_index("s")
    base = cid * HALF
    rb0 = sid * RPT

    def z16(i, _):
        zb[pl.ds(i * L, L)] = jnp.zeros((L,), f32)
        return 0

    lax.fori_loop(0, RPT // L, z16, 0)
    pltpu.sync_copy(zb, histr.at[pl.ds(rb0, RPT)])
    pltpu.sync_copy(zb, histc.at[pl.ds(rb0, RPT)])
    for k in range(CH // L):
        ones[pl.ds(k * L, L)] = jnp.ones((L,), f32)
    plsc.subcore_barrier()

    def outer(o, _):
        eb_ = sid * EPT + o * (GP * CH)
        pltpu.sync_copy(row_hbm.at[pl.ds(eb_, GP * CH)], rst)
        pltpu.sync_copy(col_hbm.at[pl.ds(eb_, GP * CH)], cst)

        def inner(g2, _):
            for k in range(CH // L):
                r = rst[pl.ds(g2 * CH + k * L, L)]
                ro = (r >= base) & (r < base + HALF)
                lr[g2, pl.ds(k * L, L)] = jnp.where(
                    ro, r - base, TRASH + (r & 7))
                c = cst[pl.ds(g2 * CH + k * L, L)]
                co = (c >= base) & (c < base + HALF)
                lc[g2, pl.ds(k * L, L)] = jnp.where(
                    co, c - base, TRASH + (c & 7))
            return 0

        lax.fori_loop(0, GP, inner, 0)

        def scat(j, _):
            pltpu.sync_copy(ones, histr.at[lr.at[j]], add=True)
            pltpu.sync_copy(ones, histc.at[lc.at[j]], add=True)
            return 0

        lax.fori_loop(0, GP, scat, 0)
        return 0

    lax.fori_loop(0, NOUTP, outer, 0)
    plsc.subcore_barrier()

    pltpu.sync_copy(histr.at[pl.ds(rb0, RPT)], hrb)
    pltpu.sync_copy(histc.at[pl.ds(rb0, RPT)], hcb)

    def dblk(b, _):
        drb[pl.ds(b * L, L)] = _rsqrt16(hrb[pl.ds(b * L, L)])
        dcb[pl.ds(b * L, L)] = _rsqrt16(hcb[pl.ds(b * L, L)])
        return 0

    lax.fori_loop(0, RPT // L, dblk, 0)
    pltpu.sync_copy(drb, dinv_r.at[pl.ds(cid * HPAD + rb0, RPT)])
    pltpu.sync_copy(dcb, dinv_c.at[pl.ds(cid * HPAD + rb0, RPT)])

    def go(o, _):
        rowb = rb0 + o * RB
        grow = cid * HPAD + rowb
        pltpu.sync_copy(emb.at[pl.ds(grow, RB)], eb)

        def blkfn(b2, _):
            roff = b2 * L
            wc = dcb[pl.ds(o * RB + roff, L)]
            for j in range(L):
                dcv = _bcast(wc, j)
                ro = roff + j
                for q in range(D // L):
                    gb[ro, pl.ds(q * L, L)] = eb[ro, pl.ds(q * L, L)] * dcv
            return 0

        lax.fori_loop(0, RB // L, blkfn, 0)
        pltpu.sync_copy(gb, g0.at[pl.ds(grow, RB)])
        return 0

    lax.fori_loop(0, NRB, go, 0)


@functools.partial(
    pl.kernel,
    out_type=(jax.ShapeDtypeStruct((NPAD, D), f32),
              jax.ShapeDtypeStruct((NPAD, D), f32)),
    mesh=_mesh,
    compiler_params=pltpu.CompilerParams(use_tc_tiling_on_sc=False),
    scratch_types=(
        pltpu.VMEM_SHARED((HPAD, D), f32),
        pltpu.VMEM((2, GL, CH), i32),
        pltpu.VMEM((2, GL, CH), i32),
        pltpu.VMEM((2, CH, D), f32),
        pltpu.VMEM((RB, D), f32),
        pltpu.VMEM((RB, D), f32),
        pltpu.VMEM((RPT,), f32),
        pltpu.VMEM((RPT,), f32),
        pltpu.SemaphoreType.DMA,
        pltpu.SemaphoreType.DMA,
        pltpu.SemaphoreType.DMA,
    ),
)
def _layer_k(rloc, cadj, gin, sin, dr, dc, gout, sout,
             acc, lst, ast, gbuf, ab, sb, drb, dcb, gsem0, gsem1, isem):
    cid = lax.axis_index("c")
    sid = lax.axis_index("s")
    rb0 = sid * RPT

    def zrow(i, _):
        for q in range(D // L):
            ab[i, pl.ds(q * L, L)] = jnp.zeros((L,), f32)
        return 0

    lax.fori_loop(0, RB, zrow, 0)

    def zcp(o, _):
        pltpu.sync_copy(ab, acc.at[pl.ds(rb0 + o * RB, RB)])
        return 0

    lax.fori_loop(0, NRB, zcp, 0)
    plsc.subcore_barrier()

    cb0 = sid * NCHT
    pltpu.async_copy(rloc.at[cid, pl.ds(cb0, GL)], lst.at[0], isem)
    pltpu.async_copy(cadj.at[pl.ds(cb0, GL)], ast.at[0], isem)

    def outer(o, _):
        slot = lax.rem(o, 2)
        pltpu.make_async_copy(rloc.at[cid, pl.ds(cb0, GL)], lst.at[slot],
                              isem).wait()
        pltpu.make_async_copy(cadj.at[pl.ds(cb0, GL)], ast.at[slot],
                              isem).wait()

        @pl.when(o + 1 < NOUTL)
        def _():
            nb = sid * NCHT + (o + 1) * GL
            pltpu.async_copy(rloc.at[cid, pl.ds(nb, GL)], lst.at[1 - slot],
                             isem)
            pltpu.async_copy(cadj.at[pl.ds(nb, GL)], ast.at[1 - slot], isem)

        pltpu.async_copy(gin.at[ast.at[slot, 0]], gbuf.at[0], gsem0)
        pltpu.async_copy(gin.at[ast.at[slot, 1]], gbuf.at[1], gsem1)

        def wait_scatter_fire(j, gs, sem, fire):
            pltpu.make_async_copy(gin.at[ast.at[slot, j]], gbuf.at[gs],
                                  sem).wait()
            pltpu.sync_copy(gbuf.at[gs], acc.at[lst.at[slot, j]], add=True)
            if fire:
                pltpu.async_copy(gin.at[ast.at[slot, j + 2]], gbuf.at[gs], sem)

        def ej(j2, _):
            wait_scatter_fire(2 * j2, 0, gsem0, True)
            wait_scatter_fire(2 * j2 + 1, 1, gsem1, True)
            return 0

        lax.fori_loop(0, (GL - 2) // 2, ej, 0)
        wait_scatter_fire(GL - 2, 0, gsem0, False)
        wait_scatter_fire(GL - 1, 1, gsem1, False)
        return 0

    lax.fori_loop(0, NOUTL, outer, 0)
    plsc.subcore_barrier()

    pltpu.sync_copy(dr.at[pl.ds(cid * HPAD + rb0, RPT)], drb)
    pltpu.sync_copy(dc.at[pl.ds(cid * HPAD + rb0, RPT)], dcb)

    def no(o, _):
        rowb = rb0 + o * RB
        grow = cid * HPAD + rowb
        pltpu.sync_copy(acc.at[pl.ds(rowb, RB)], ab)
        pltpu.sync_copy(sin.at[pl.ds(grow, RB)], sb)

        def blkfn(b2, _):
            roff = b2 * L
            wr = drb[pl.ds(o * RB + roff, L)]
            wc = dcb[pl.ds(o * RB + roff, L)]
            for j in range(L):
                brv = _bcast(wr, j)
                brc = brv * _bcast(wc, j)
                ro = roff + j
                for q in range(D // L):
                    s = ab[ro, pl.ds(q * L, L)]
                    ab[ro, pl.ds(q * L, L)] = s * brc
                    sb[ro, pl.ds(q * L, L)] = sb[ro, pl.ds(q * L, L)] + s * brv
            return 0

        lax.fori_loop(0, RB // L, blkfn, 0)
        pltpu.sync_copy(ab, gout.at[pl.ds(grow, RB)])
        pltpu.sync_copy(sb, sout.at[pl.ds(grow, RB)])
        return 0

    lax.fori_loop(0, NRB, no, 0)


@functools.partial(
    pl.kernel,
    out_type=jax.ShapeDtypeStruct((NBATCH,), f32),
    mesh=_mesh,
    compiler_params=pltpu.CompilerParams(use_tc_tiling_on_sc=False),
    scratch_types=(
        pltpu.VMEM((BPT,), i32),
        pltpu.VMEM((BPT,), i32),
        pltpu.VMEM((BPT // CH, CH), i32),
        pltpu.VMEM((BPT // CH, CH), i32),
        pltpu.VMEM((CH, D), f32),
        pltpu.VMEM((CH, D), f32),
        pltpu.VMEM((BPT,), f32),
        pltpu.VMEM((2 * L,), f32),
    ),
)
def _gamma_k(users, items, sfin, gamma, ust, ist, ub2, ib2, ur, ir, ob, tb):
    cid = lax.axis_index("c")
    sid = lax.axis_index("s")
    wid = cid * NS + sid
    bb = wid * BPT
    pltpu.sync_copy(users.at[pl.ds(bb, BPT)], ust)
    pltpu.sync_copy(items.at[pl.ds(bb, BPT)], ist)

    def re(i, _):
        j2 = i // (CH // L)
        k = i % (CH // L)
        ub2[j2, pl.ds(k * L, L)] = ust[pl.ds(i * L, L)]
        ib2[j2, pl.ds(k * L, L)] = ist[pl.ds(i * L, L)] + HPAD
        return 0

    lax.fori_loop(0, BPT // L, re, 0)
    lane = lax.iota(i32, L)
    tb[pl.ds(L, L)] = jnp.zeros((L,), f32)

    def jf(j, _):
        pltpu.sync_copy(sfin.at[ub2.at[j]], ur)
        pltpu.sync_copy(sfin.at[ib2.at[j]], ir)

        def grp(g, _):
            out16 = jnp.zeros((L,), f32)
            for e in range(L):
                ro = g * L + e
                acc = ur[ro, pl.ds(0, L)] * ir[ro, pl.ds(0, L)]
                for q in range(1, D // L):
                    acc = acc + ur[ro, pl.ds(q * L, L)] * ir[ro, pl.ds(q * L, L)]
                for sh in (8, 4, 2, 1):
                    tb[pl.ds(0, L)] = acc
                    acc = acc + tb[pl.ds(sh, L)]
                dsum = _bcast(acc, 0) * (1.0 / 25.0)
                out16 = jnp.where(lane == e, dsum, out16)
            ob[pl.ds(j * CH + g * L, L)] = out16
            return 0

        lax.fori_loop(0, CH // L, grp, 0)
        return 0

    lax.fori_loop(0, BPT // CH, jf, 0)
    pltpu.sync_copy(ob, gamma.at[pl.ds(bb, BPT)])


def kernel(users, items, user_emb, item_emb, edge_index):
    row = edge_index[0].astype(i32)
    col = edge_index[1].astype(i32)
    pad = jnp.full((EPAD - E,), SENT, i32)
    row_p = jnp.concatenate([row, pad])
    col_p = jnp.concatenate([col, pad])
    zpad = jnp.zeros((HPAD - HALF, D), f32)
    emb = jnp.concatenate([user_emb, zpad, item_emb, zpad], axis=0)
    rloc, cadj = _prep_k(row_p, col_p)
    dinv_r, dinv_c, g0 = _deg_k(row_p, col_p, emb)
    g, s = g0, emb
    for _ in range(LAYERS):
        g, s = _layer_k(rloc, cadj, g, s, dinv_r, dinv_c)
    return _gamma_k(users.astype(i32), items.astype(i32), s)


# speedup vs baseline: 2.2609x; 1.0025x over previous
import functools

import jax
import jax.numpy as jnp
from jax import lax
from jax.experimental import pallas as pl
from jax.experimental.pallas import tpu as pltpu
from jax.experimental.pallas import tpu_sc as plsc

f32 = jnp.float32
i32 = jnp.int32

NU, NI = 25000, 25000
N = NU + NI
D = 64
E = 800000
NBATCH = 16384
LAYERS = 4
NC, NS, L = 2, 16, 16
HALF = 25000
HPAD = 25088
RPT = HPAD // NS
NPAD = 2 * HPAD
TRASH = HALF
CH = 128
NCHT = 392
EPT = NCHT * CH
EPAD = NS * EPT
GP = 56
NOUTP = NCHT // GP
GL = 8
NOUTL = NCHT // GL
SENT = 1 << 30
RB = 32
NRB = RPT // RB
BPT = NBATCH // (NC * NS)

_mesh = plsc.VectorSubcoreMesh(core_axis_name="c", subcore_axis_name="s",
                               num_cores=NC, num_subcores=NS)


def _rsqrt16(x):
    x = jnp.maximum(x, 1.0)
    ii = lax.bitcast_convert_type(x, i32)
    ii = 0x5F3759DF - lax.shift_right_logical(ii, 1)
    y = lax.bitcast_convert_type(ii, f32)
    for _ in range(3):
        y = y * (1.5 - 0.5 * x * y * y)
    return y


def _bcast(w, j):
    return lax.broadcast_in_dim(lax.slice_in_dim(w, j, j + 1), (L,), (0,))


@functools.partial(
    pl.kernel,
    out_type=(jax.ShapeDtypeStruct((NC, NS * NCHT, CH), i32),
              jax.ShapeDtypeStruct((NS * NCHT, CH), i32)),
    mesh=_mesh,
    compiler_params=pltpu.CompilerParams(use_tc_tiling_on_sc=False),
    scratch_types=(
        pltpu.VMEM((GP * CH,), i32),
        pltpu.VMEM((GP * CH,), i32),
        pltpu.VMEM((GP, CH), i32),
        pltpu.VMEM((GP, CH), i32),
    ),
)
def _prep_k(row_hbm, col_hbm, rloc, cadj, rst, cst, lst, ast):
    cid = lax.axis_index("c")
    sid = lax.axis_index("s")
    base = cid * HALF

    def outer(o, _):
        eb = sid * EPT + o * (GP * CH)
        pltpu.sync_copy(row_hbm.at[pl.ds(eb, GP * CH)], rst)
        pltpu.sync_copy(col_hbm.at[pl.ds(eb, GP * CH)], cst)

        def inner(g2, _):
            for k in range(CH // L):
                r = rst[pl.ds(g2 * CH + k * L, L)]
                owned = (r >= base) & (r < base + HALF)
                lst[g2, pl.ds(k * L, L)] = jnp.where(
                    owned, r - base, TRASH + (r & 7))
                c = cst[pl.ds(g2 * CH + k * L, L)]
                cc = jnp.where(c >= HALF, c + (HPAD - HALF), c)
                ast[g2, pl.ds(k * L, L)] = jnp.where(c >= SENT // 2, 0, cc)
            return 0

        lax.fori_loop(0, GP, inner, 0)
        rb = sid * NCHT + o * GP
        pltpu.sync_copy(lst, rloc.at[cid, pl.ds(rb, GP)])

        @pl.when(cid == 0)
        def _():
            pltpu.sync_copy(ast, cadj.at[pl.ds(rb, GP)])

        return 0

    lax.fori_loop(0, NOUTP, outer, 0)


@functools.partial(
    pl.kernel,
    out_type=(jax.ShapeDtypeStruct((NPAD,), f32),
              jax.ShapeDtypeStruct((NPAD,), f32),
              jax.ShapeDtypeStruct((NPAD, D), f32)),
    mesh=_mesh,
    compiler_params=pltpu.CompilerParams(use_tc_tiling_on_sc=False),
    scratch_types=(
        pltpu.VMEM_SHARED((HPAD,), f32),
        pltpu.VMEM_SHARED((HPAD,), f32),
        pltpu.VMEM((GP * CH,), i32),
        pltpu.VMEM((GP * CH,), i32),
        pltpu.VMEM((GP, CH), i32),
        pltpu.VMEM((GP, CH), i32),
        pltpu.VMEM((CH,), f32),
        pltpu.VMEM((RPT,), f32),
        pltpu.VMEM((RPT,), f32),
        pltpu.VMEM((RPT,), f32),
        pltpu.VMEM((RPT,), f32),
        pltpu.VMEM((RPT,), f32),
        pltpu.VMEM((RB, D), f32),
        pltpu.VMEM((RB, D), f32),
        pltpu.SemaphoreType.DMA,
    ),
)
def _deg_k(row_hbm, col_hbm, emb, dinv_r, dinv_c, g0,
           histr, histc, rst, cst, lr, lc, ones, zb, hrb, hcb, drb, dcb,
           eb, gb, hsem):
    cid = lax.axis_index("c")
    sid = lax.axis_index("s")
    base = cid * HALF
    rb0 = sid * RPT

    def z16(i, _):
        zb[pl.ds(i * L, L)] = jnp.zeros((L,), f32)
        return 0

    lax.fori_loop(0, RPT // L, z16, 0)
    pltpu.sync_copy(zb, histr.at[pl.ds(rb0, RPT)])
    pltpu.sync_copy(zb, histc.at[pl.ds(rb0, RPT)])
    for k in range(CH // L):
        ones[pl.ds(k * L, L)] = jnp.ones((L,), f32)
    plsc.subcore_barrier()

    def outer(o, _):
        eb_ = sid * EPT + o * (GP * CH)
        pltpu.sync_copy(row_hbm.at[pl.ds(eb_, GP * CH)], rst)
        pltpu.sync_copy(col_hbm.at[pl.ds(eb_, GP * CH)], cst)

        def inner(g2, _):
            for k in range(CH // L):
                r = rst[pl.ds(g2 * CH + k * L, L)]
                ro = (r >= base) & (r < base + HALF)
                lr[g2, pl.ds(k * L, L)] = jnp.where(
                    ro, r - base, TRASH + (r & 7))
                c = cst[pl.ds(g2 * CH + k * L, L)]
                co = (c >= base) & (c < base + HALF)
                lc[g2, pl.ds(k * L, L)] = jnp.where(
                    co, c - base, TRASH + (c & 7))
            return 0

        lax.fori_loop(0, GP, inner, 0)

        def scat(j, _):
            pltpu.async_copy(ones, histr.at[lr.at[j]], hsem, add=True)
            pltpu.async_copy(ones, histc.at[lc.at[j]], hsem, add=True)
            return 0

        lax.fori_loop(0, GP, scat, 0)

        def drain(j, _):
            pltpu.make_async_copy(ones, histr.at[lr.at[j]], hsem).wait()
            pltpu.make_async_copy(ones, histc.at[lc.at[j]], hsem).wait()
            return 0

        lax.fori_loop(0, GP, drain, 0)
        return 0

    lax.fori_loop(0, NOUTP, outer, 0)
    plsc.subcore_barrier()

    pltpu.sync_copy(histr.at[pl.ds(rb0, RPT)], hrb)
    pltpu.sync_copy(histc.at[pl.ds(rb0, RPT)], hcb)

    def dblk(b, _):
        drb[pl.ds(b * L, L)] = _rsqrt16(hrb[pl.ds(b * L, L)])
        dcb[pl.ds(b * L, L)] = _rsqrt16(hcb[pl.ds(b * L, L)])
        return 0

    lax.fori_loop(0, RPT // L, dblk, 0)
    pltpu.sync_copy(drb, dinv_r.at[pl.ds(cid * HPAD + rb0, RPT)])
    pltpu.sync_copy(dcb, dinv_c.at[pl.ds(cid * HPAD + rb0, RPT)])

    def go(o, _):
        rowb = rb0 + o * RB
        grow = cid * HPAD + rowb
        pltpu.sync_copy(emb.at[pl.ds(grow, RB)], eb)

        def blkfn(b2, _):
            roff = b2 * L
            wc = dcb[pl.ds(o * RB + roff, L)]
            for j in range(L):
                dcv = _bcast(wc, j)
                ro = roff + j
                for q in range(D // L):
                    gb[ro, pl.ds(q * L, L)] = eb[ro, pl.ds(q * L, L)] * dcv
            return 0

        lax.fori_loop(0, RB // L, blkfn, 0)
        pltpu.sync_copy(gb, g0.at[pl.ds(grow, RB)])
        return 0

    lax.fori_loop(0, NRB, go, 0)


@functools.partial(
    pl.kernel,
    out_type=(jax.ShapeDtypeStruct((NPAD, D), f32),
              jax.ShapeDtypeStruct((NPAD, D), f32)),
    mesh=_mesh,
    compiler_params=pltpu.CompilerParams(use_tc_tiling_on_sc=False),
    scratch_types=(
        pltpu.VMEM_SHARED((HPAD, D), f32),
        pltpu.VMEM((2, GL, CH), i32),
        pltpu.VMEM((2, GL, CH), i32),
        pltpu.VMEM((2, CH, D), f32),
        pltpu.VMEM((RB, D), f32),
        pltpu.VMEM((RB, D), f32),
        pltpu.VMEM((RPT,), f32),
        pltpu.VMEM((RPT,), f32),
        pltpu.SemaphoreType.DMA,
        pltpu.SemaphoreType.DMA,
        pltpu.SemaphoreType.DMA,
    ),
)
def _layer_k(rloc, cadj, gin, sin, dr, dc, gout, sout,
             acc, lst, ast, gbuf, ab, sb, drb, dcb, gsem0, gsem1, isem):
    cid = lax.axis_index("c")
    sid = lax.axis_index("s")
    rb0 = sid * RPT

    def zrow(i, _):
        for q in range(D // L):
            ab[i, pl.ds(q * L, L)] = jnp.zeros((L,), f32)
        return 0

    lax.fori_loop(0, RB, zrow, 0)

    def zcp(o, _):
        pltpu.sync_copy(ab, acc.at[pl.ds(rb0 + o * RB, RB)])
        return 0

    lax.fori_loop(0, NRB, zcp, 0)
    plsc.subcore_barrier()

    cb0 = sid * NCHT
    pltpu.async_copy(rloc.at[cid, pl.ds(cb0, GL)], lst.at[0], isem)
    pltpu.async_copy(cadj.at[pl.ds(cb0, GL)], ast.at[0], isem)

    def outer(o, _):
        slot = lax.rem(o, 2)
        pltpu.make_async_copy(rloc.at[cid, pl.ds(cb0, GL)], lst.at[slot],
                              isem).wait()
        pltpu.make_async_copy(cadj.at[pl.ds(cb0, GL)], ast.at[slot],
                              isem).wait()

        @pl.when(o + 1 < NOUTL)
        def _():
            nb = sid * NCHT + (o + 1) * GL
            pltpu.async_copy(rloc.at[cid, pl.ds(nb, GL)], lst.at[1 - slot],
                             isem)
            pltpu.async_copy(cadj.at[pl.ds(nb, GL)], ast.at[1 - slot], isem)

        pltpu.async_copy(gin.at[ast.at[slot, 0]], gbuf.at[0], gsem0)
        pltpu.async_copy(gin.at[ast.at[slot, 1]], gbuf.at[1], gsem1)

        def wait_scatter_fire(j, gs, sem, fire):
            pltpu.make_async_copy(gin.at[ast.at[slot, j]], gbuf.at[gs],
                                  sem).wait()
            pltpu.sync_copy(gbuf.at[gs], acc.at[lst.at[slot, j]], add=True)
            if fire:
                pltpu.async_copy(gin.at[ast.at[slot, j + 2]], gbuf.at[gs], sem)

        def ej(j2, _):
            wait_scatter_fire(2 * j2, 0, gsem0, True)
            wait_scatter_fire(2 * j2 + 1, 1, gsem1, True)
            return 0

        lax.fori_loop(0, (GL - 2) // 2, ej, 0)
        wait_scatter_fire(GL - 2, 0, gsem0, False)
        wait_scatter_fire(GL - 1, 1, gsem1, False)
        return 0

    lax.fori_loop(0, NOUTL, outer, 0)
    plsc.subcore_barrier()

    pltpu.sync_copy(dr.at[pl.ds(cid * HPAD + rb0, RPT)], drb)
    pltpu.sync_copy(dc.at[pl.ds(cid * HPAD + rb0, RPT)], dcb)

    def no(o, _):
        rowb = rb0 + o * RB
        grow = cid * HPAD + rowb
        pltpu.sync_copy(acc.at[pl.ds(rowb, RB)], ab)
        pltpu.sync_copy(sin.at[pl.ds(grow, RB)], sb)

        def blkfn(b2, _):
            roff = b2 * L
            wr = drb[pl.ds(o * RB + roff, L)]
            wc = dcb[pl.ds(o * RB + roff, L)]
            for j in range(L):
                brv = _bcast(wr, j)
                brc = brv * _bcast(wc, j)
                ro = roff + j
                for q in range(D // L):
                    s = ab[ro, pl.ds(q * L, L)]
                    ab[ro, pl.ds(q * L, L)] = s * brc
                    sb[ro, pl.ds(q * L, L)] = sb[ro, pl.ds(q * L, L)] + s * brv
            return 0

        lax.fori_loop(0, RB // L, blkfn, 0)
        pltpu.sync_copy(ab, gout.at[pl.ds(grow, RB)])
        pltpu.sync_copy(sb, sout.at[pl.ds(grow, RB)])
        return 0

    lax.fori_loop(0, NRB, no, 0)


@functools.partial(
    pl.kernel,
    out_type=jax.ShapeDtypeStruct((NBATCH,), f32),
    mesh=_mesh,
    compiler_params=pltpu.CompilerParams(use_tc_tiling_on_sc=False),
    scratch_types=(
        pltpu.VMEM((BPT,), i32),
        pltpu.VMEM((BPT,), i32),
        pltpu.VMEM((BPT // CH, CH), i32),
        pltpu.VMEM((BPT // CH, CH), i32),
        pltpu.VMEM((CH, D), f32),
        pltpu.VMEM((CH, D), f32),
        pltpu.VMEM((BPT,), f32),
        pltpu.VMEM((2 * L,), f32),
    ),
)
def _gamma_k(users, items, sfin, gamma, ust, ist, ub2, ib2, ur, ir, ob, tb):
    cid = lax.axis_index("c")
    sid = lax.axis_index("s")
    wid = cid * NS + sid
    bb = wid * BPT
    pltpu.sync_copy(users.at[pl.ds(bb, BPT)], ust)
    pltpu.sync_copy(items.at[pl.ds(bb, BPT)], ist)

    def re(i, _):
        j2 = i // (CH // L)
        k = i % (CH // L)
        ub2[j2, pl.ds(k * L, L)] = ust[pl.ds(i * L, L)]
        ib2[j2, pl.ds(k * L, L)] = ist[pl.ds(i * L, L)] + HPAD
        return 0

    lax.fori_loop(0, BPT // L, re, 0)
    lane = lax.iota(i32, L)
    tb[pl.ds(L, L)] = jnp.zeros((L,), f32)

    def jf(j, _):
        pltpu.sync_copy(sfin.at[ub2.at[j]], ur)
        pltpu.sync_copy(sfin.at[ib2.at[j]], ir)

        def grp(g, _):
            out16 = jnp.zeros((L,), f32)
            for e in range(L):
                ro = g * L + e
                acc = ur[ro, pl.ds(0, L)] * ir[ro, pl.ds(0, L)]
                for q in range(1, D // L):
                    acc = acc + ur[ro, pl.ds(q * L, L)] * ir[ro, pl.ds(q * L, L)]
                for sh in (8, 4, 2, 1):
                    tb[pl.ds(0, L)] = acc
                    acc = acc + tb[pl.ds(sh, L)]
                dsum = _bcast(acc, 0) * (1.0 / 25.0)
                out16 = jnp.where(lane == e, dsum, out16)
            ob[pl.ds(j * CH + g * L, L)] = out16
            return 0

        lax.fori_loop(0, CH // L, grp, 0)
        return 0

    lax.fori_loop(0, BPT // CH, jf, 0)
    pltpu.sync_copy(ob, gamma.at[pl.ds(bb, BPT)])


def kernel(users, items, user_emb, item_emb, edge_index):
    row = edge_index[0].astype(i32)
    col = edge_index[1].astype(i32)
    pad = jnp.full((EPAD - E,), SENT, i32)
    row_p = jnp.concatenate([row, pad])
    col_p = jnp.concatenate([col, pad])
    zpad = jnp.zeros((HPAD - HALF, D), f32)
    emb = jnp.concatenate([user_emb, zpad, item_emb, zpad], axis=0)
    rloc, cadj = _prep_k(row_p, col_p)
    dinv_r, dinv_c, g0 = _deg_k(row_p, col_p, emb)
    g, s = g0, emb
    for _ in range(LAYERS):
        g, s = _layer_k(rloc, cadj, g, s, dinv_r, dinv_c)
    return _gamma_k(users.astype(i32), items.astype(i32), s)
